# baseline (speedup 1.0000x reference)
import functools

import jax
import jax.numpy as jnp
from jax import lax
from jax.experimental import pallas as pl
from jax.experimental.pallas import tpu as pltpu
from jax.experimental.pallas import tpu_sc as plsc

BATCH = 16384
DIM = 64
NUSERS = 1000000
NC, NS = 2, 16
NW = NC * NS
B_PER_W = BATCH // NW
CHUNK = 128
N_CHUNKS = B_PER_W // CHUNK
PROWS = NUSERS // 2


def _sc_gather(up2, ip2, gu_p, gi_p, mu_p, mi_p):
    mesh = plsc.VectorSubcoreMesh(core_axis_name="c", subcore_axis_name="s")

    @functools.partial(
        pl.kernel,
        out_type=[jax.ShapeDtypeStruct((BATCH, 128), jnp.float32)] * 4,
        mesh=mesh,
        scratch_types=[
            pltpu.VMEM((N_CHUNKS, CHUNK), jnp.int32),
            pltpu.VMEM((N_CHUNKS, CHUNK), jnp.int32),
            pltpu.VMEM((CHUNK, 128), jnp.float32),
            pltpu.VMEM((CHUNK, 128), jnp.float32),
            pltpu.SemaphoreType.DMA,
            pltpu.SemaphoreType.DMA,
        ],
    )
    def k(up_hbm, ip_hbm, gu_hbm, gi_hbm, mu_hbm, mi_hbm,
          gu_out, gi_out, mu_out, mi_out,
          up_v, ip_v, buf_a, buf_b, sem_a, sem_b):
        wid = lax.axis_index("s") * NC + lax.axis_index("c")
        crow = wid * N_CHUNKS
        base = wid * B_PER_W
        pltpu.sync_copy(up_hbm.at[pl.ds(crow, N_CHUNKS)], up_v)
        pltpu.sync_copy(ip_hbm.at[pl.ds(crow, N_CHUNKS)], ip_v)

        jobs = []
        for table, idx_v, out in ((gu_hbm, up_v, gu_out),
                                  (gi_hbm, ip_v, gi_out),
                                  (mu_hbm, up_v, mu_out),
                                  (mi_hbm, ip_v, mi_out)):
            for j in range(N_CHUNKS):
                jobs.append((table, idx_v, out, j))

        bufs = (buf_a, buf_b)
        sems = (sem_a, sem_b)
        prev = None
        for n, (table, idx_v, out, j) in enumerate(jobs):
            s = n % 2
            cp = pltpu.async_copy(table.at[idx_v.at[j]], bufs[s], sems[s])
            if prev is not None:
                p_s, p_out, p_off, p_cp = prev
                p_cp.wait()
                pltpu.sync_copy(bufs[p_s], p_out.at[pl.ds(p_off, CHUNK)])
            prev = (s, out, base + j * CHUNK, cp)
        p_s, p_out, p_off, p_cp = prev
        p_cp.wait()
        pltpu.sync_copy(bufs[p_s], p_out.at[pl.ds(p_off, CHUNK)])

    return k(up2, ip2, gu_p, gi_p, mu_p, mi_p)


BM = 2048


def _sel_half(slab_ref, sel2):
    x = slab_ref[...]
    m0 = (sel2[:, 0:1] != 0).astype(jnp.float32)
    m1 = (sel2[:, 1:2] != 0).astype(jnp.float32)
    return x[:, :DIM] * m0 + x[:, DIM:] * m1


def _tc_mlp(gu_s, gi_s, mu_s, mi_s, selu2, seli2, W1, b1, W2, b2, Wf, bf):
    def body(gu_ref, gi_ref, mu_ref, mi_ref, selu_ref, seli_ref,
             w1_ref, b1_ref, w2_ref, b2_ref, wf_ref, bf_ref, out_ref):
        su = selu_ref[...]
        si = seli_ref[...]
        gmf = _sel_half(gu_ref, su) * _sel_half(gi_ref, si)
        mu = _sel_half(mu_ref, su)
        mi = _sel_half(mi_ref, si)
        f32 = jnp.float32
        w1 = w1_ref[...]
        h = (jnp.dot(mu, w1[:DIM], preferred_element_type=f32)
             + jnp.dot(mi, w1[DIM:], preferred_element_type=f32))
        h = jnp.maximum(h + b1_ref[...], 0.0)
        h = jnp.maximum(
            jnp.dot(h, w2_ref[...], preferred_element_type=f32)
            + b2_ref[...], 0.0)
        wf = wf_ref[...]
        pred = (jnp.dot(gmf, wf[:DIM], preferred_element_type=f32)
                + jnp.dot(h, wf[DIM:], preferred_element_type=f32)
                + bf_ref[...])
        out_ref[...] = pred

    grid = (BATCH // BM,)
    slab_spec = pl.BlockSpec((BM, 128), lambda i: (i, 0))
    sel_spec = pl.BlockSpec((BM, 2), lambda i: (i, 0))
    full = lambda shape: pl.BlockSpec(shape, lambda i: (0,) * len(shape))
    return pl.pallas_call(
        body,
        grid=grid,
        in_specs=[
            slab_spec, slab_spec, slab_spec, slab_spec,
            sel_spec, sel_spec,
            full((2 * DIM, DIM)), full((1, DIM)),
            full((DIM, 32)), full((1, 32)),
            full((DIM + 32, 1)), full((1, 1)),
        ],
        out_specs=pl.BlockSpec((BM, 1), lambda i: (i, 0)),
        out_shape=jax.ShapeDtypeStruct((BATCH, 1), jnp.float32),
    )(gu_s, gi_s, mu_s, mi_s, selu2, seli2, W1, b1, W2, b2, Wf, bf)


def _onehot(v, n):
    return (jnp.arange(n, dtype=jnp.int32)[None, :]
            == v[:, None]).astype(jnp.int32)


def kernel(user_ids, item_ids, gmf_user_w, gmf_item_w, mlp_user_w, mlp_item_w,
           W1, b1, W2, b2, Wf, bf):
    uidx = user_ids.astype(jnp.int32)
    iidx = item_ids.astype(jnp.int32)
    shp = (BATCH // CHUNK, CHUNK)
    up2, ip2 = (uidx // 2).reshape(shp), (iidx // 2).reshape(shp)
    selu2, seli2 = _onehot(uidx % 2, 2), _onehot(iidx % 2, 2)

    one = (bf * 0.0 + 1.0)[0]
    gu_s, gi_s, mu_s, mi_s = _sc_gather(
        up2, ip2,
        (gmf_user_w * one).reshape(PROWS, 128),
        (gmf_item_w * one).reshape(PROWS, 128),
        (mlp_user_w * one).reshape(PROWS, 128),
        (mlp_item_w * one).reshape(PROWS, 128))

    pred = _tc_mlp(gu_s, gi_s, mu_s, mi_s, selu2, seli2,
                   W1, b1.reshape(1, DIM), W2, b2.reshape(1, 32),
                   Wf, bf.reshape(1, 1))
    return pred[:, 0]


# speedup vs baseline: 2.5408x; 2.5408x over previous
import functools

import jax
import jax.numpy as jnp
from jax import lax
from jax.experimental import pallas as pl
from jax.experimental.pallas import tpu as pltpu
from jax.experimental.pallas import tpu_sc as plsc

BATCH = 16384
DIM = 64
NC, NS = 2, 16
NW = NC * NS
B_PER_W = BATCH // NW


def _sc_gather(uidx, iidx, gu, gi, mu, mi):
    mesh = plsc.VectorSubcoreMesh(core_axis_name="c", subcore_axis_name="s")
    HALF = B_PER_W // 2

    @functools.partial(
        pl.kernel,
        out_type=[jax.ShapeDtypeStruct((BATCH, DIM), jnp.float32)] * 4,
        mesh=mesh,
        scratch_types=[
            pltpu.VMEM((B_PER_W,), jnp.int32),
            pltpu.VMEM((B_PER_W,), jnp.int32),
            pltpu.VMEM((HALF, DIM), jnp.float32),
            pltpu.VMEM((HALF, DIM), jnp.float32),
            pltpu.SemaphoreType.DMA,
            pltpu.SemaphoreType.DMA,
        ],
    )
    def k(uidx_hbm, iidx_hbm, gu_hbm, gi_hbm, mu_hbm, mi_hbm,
          gu_out, gi_out, mu_out, mi_out,
          uidx_v, iidx_v, buf_a, buf_b, sem_a, sem_b):
        wid = lax.axis_index("s") * NC + lax.axis_index("c")
        base = wid * B_PER_W
        pltpu.sync_copy(uidx_hbm.at[pl.ds(base, B_PER_W)], uidx_v)
        pltpu.sync_copy(iidx_hbm.at[pl.ds(base, B_PER_W)], iidx_v)

        jobs = []
        for table, idx_v, out in ((gu_hbm, uidx_v, gu_out),
                                  (gi_hbm, iidx_v, gi_out),
                                  (mu_hbm, uidx_v, mu_out),
                                  (mi_hbm, iidx_v, mi_out)):
            jobs.append((table, idx_v, out, 0))
            jobs.append((table, idx_v, out, 1))

        bufs = (buf_a, buf_b)
        sems = (sem_a, sem_b)

        def fire(table, idx_v, buf, sem, h):
            def body(g, _):
                vec = idx_v[pl.ds(h * HALF + g * 16, 16)]
                for lane in range(16):
                    i = vec[lane]
                    pltpu.async_copy(table.at[pl.ds(i, 1)],
                                     buf.at[pl.ds(g * 16 + lane, 1)], sem)
                return _
            lax.fori_loop(0, HALF // 16, body, 0)

        def drain_and_write(n):
            table, idx_v, out, h = jobs[n]
            s = n % 2
            pltpu.make_async_copy(out.at[pl.ds(0, HALF)], bufs[s],
                                  sems[s]).wait()
            pltpu.sync_copy(bufs[s],
                            out.at[pl.ds(base + h * HALF, HALF)])

        for n, (table, idx_v, out, h) in enumerate(jobs):
            if n >= 2:
                drain_and_write(n - 2)
            fire(table, idx_v, bufs[n % 2], sems[n % 2], h)
        drain_and_write(6)
        drain_and_write(7)

    return k(uidx, iidx, gu, gi, mu, mi)


BM = 2048


def _tc_mlp(gu_rows, gi_rows, mu_rows, mi_rows, W1, b1, W2, b2, Wf, bf):
    def body(gu_ref, gi_ref, mu_ref, mi_ref,
             w1_ref, b1_ref, w2_ref, b2_ref, wf_ref, bf_ref, out_ref):
        gmf = gu_ref[...] * gi_ref[...]
        w1 = w1_ref[...]
        h = jnp.dot(mu_ref[...], w1[:DIM], preferred_element_type=jnp.float32)
        h = h + jnp.dot(mi_ref[...], w1[DIM:],
                        preferred_element_type=jnp.float32)
        h = jnp.maximum(h + b1_ref[...], 0.0)
        h = jnp.maximum(
            jnp.dot(h, w2_ref[...], preferred_element_type=jnp.float32)
            + b2_ref[...], 0.0)
        wf = wf_ref[...]
        pred = (jnp.dot(gmf, wf[:DIM], preferred_element_type=jnp.float32)
                + jnp.dot(h, wf[DIM:], preferred_element_type=jnp.float32)
                + bf_ref[...])
        out_ref[...] = pred

    grid = (BATCH // BM,)
    rows_spec = pl.BlockSpec((BM, DIM), lambda i: (i, 0))
    full = lambda shape: pl.BlockSpec(shape, lambda i: (0,) * len(shape))
    return pl.pallas_call(
        body,
        grid=grid,
        in_specs=[
            rows_spec, rows_spec, rows_spec, rows_spec,
            full((2 * DIM, DIM)), full((1, DIM)),
            full((DIM, 32)), full((1, 32)),
            full((DIM + 32, 1)), full((1, 1)),
        ],
        out_specs=pl.BlockSpec((BM, 1), lambda i: (i, 0)),
        out_shape=jax.ShapeDtypeStruct((BATCH, 1), jnp.float32),
    )(gu_rows, gi_rows, mu_rows, mi_rows, W1, b1, W2, b2, Wf, bf)


def kernel(user_ids, item_ids, gmf_user_w, gmf_item_w, mlp_user_w, mlp_item_w,
           W1, b1, W2, b2, Wf, bf):
    uidx = user_ids.astype(jnp.int32)
    iidx = item_ids.astype(jnp.int32)
    gu, gi, mu, mi = _sc_gather(uidx, iidx,
                                gmf_user_w, gmf_item_w, mlp_user_w, mlp_item_w)
    pred = _tc_mlp(gu, gi, mu, mi,
                   W1, b1.reshape(1, DIM), W2, b2.reshape(1, 32),
                   Wf, bf.reshape(1, 1))
    return pred[:, 0]
